# baseline (speedup 1.0000x reference)
import functools

import jax
import jax.numpy as jnp
from jax import lax
from jax.experimental import pallas as pl
from jax.experimental.pallas import tpu as pltpu
from jax.experimental.pallas import tpu_sc as plsc

EPS = 1e-5

B, N, K, C = 4, 4096, 16, 128
BN = B * N
R = BN * K
TP = 512
RT = TP * K
NT = BN // TP
CH = 128


def _sc_gather_body(nw, tab_hbm, idx_hbm, out_hbm, idxv, rows, g0, g1):
    rw = (2 * R) // nw
    nc = plsc.get_sparse_core_info().num_cores
    wid = lax.axis_index("s") * nc + lax.axis_index("c")
    base = wid * rw
    pltpu.sync_copy(idx_hbm.at[pl.ds(base, rw)], idxv)
    nj = (rw // CH) // 2
    H = CH // 2

    def gath(i, buf, sem):
        pltpu.async_copy(
            tab_hbm.at[idxv.at[pl.ds(i * CH, CH)]], rows.at[buf], sem)

    def gwait(i, buf, sem):
        pltpu.make_async_copy(
            tab_hbm.at[idxv.at[pl.ds(i * CH, CH)]], rows.at[buf], sem).wait()

    def wback(i, buf):
        ob = (base + i * CH) // 2
        pltpu.sync_copy(rows.at[buf, pl.ds(0, H)],
                        out_hbm.at[pl.ds(ob, H), pl.ds(0, C // 2)])
        pltpu.sync_copy(rows.at[buf, pl.ds(H, H)],
                        out_hbm.at[pl.ds(ob, H), pl.ds(C // 2, C // 2)])

    gath(0, 0, g0)

    def body(j, carry):
        i0 = 2 * j
        gwait(i0, 0, g0)
        gath(i0 + 1, 1, g1)
        wback(i0, 0)

        @pl.when(j + 1 < nj)
        def _():
            gath(i0 + 2, 0, g0)

        gwait(i0 + 1, 1, g1)
        wback(i0 + 1, 1)
        return carry

    lax.fori_loop(0, nj, body, 0)


def _sc_gather(tab, idxg):
    info = plsc.get_sparse_core_info()
    nw = info.num_cores * info.num_subcores
    rw = (2 * R) // nw
    mesh = plsc.VectorSubcoreMesh(core_axis_name="c", subcore_axis_name="s")
    f = pl.kernel(
        functools.partial(_sc_gather_body, nw),
        out_type=jax.ShapeDtypeStruct((R, C), jnp.int32),
        mesh=mesh,
        scratch_types=[
            pltpu.VMEM((rw,), jnp.int32),
            pltpu.VMEM((2, CH, C // 2), jnp.int32),
            pltpu.SemaphoreType.DMA,
            pltpu.SemaphoreType.DMA,
        ],
        compiler_params=pltpu.CompilerParams(use_tc_tiling_on_sc=False),
    )
    return f(tab, idxg)


RT2 = RT // 2


def _lohi(Gp):
    lo = ((Gp << 16) >> 16).astype(jnp.float32)
    hi = (Gp >> 16).astype(jnp.float32)
    return lo, hi


def _rep8(x, w):
    return jnp.broadcast_to(x[:, None, :], (TP, K // 2, w)).reshape(RT2, w)


def _p1_body(Gp_ref, F_ref, FED_ref, FOD_ref, w1t_ref, b1_ref,
             sw1tp_ref, sb1_ref, W1E2_ref, W1O2_ref, HALF8_ref, HALFT8_ref,
             FOLD_ref, WN_ref, DST_ref, SST_ref):
    t = pl.program_id(1)
    lo, hi = _lohi(Gp_ref[0])
    F = F_ref[0]

    FW = jnp.dot(F, w1t_ref[...], preferred_element_type=jnp.float32, precision=lax.Precision.HIGHEST) + b1_ref[...]
    FWpair = _rep8(jnp.concatenate([FW, FW], axis=1), 2 * C)
    GW = (jnp.dot(lo, W1E2_ref[...], preferred_element_type=jnp.float32)
          + jnp.dot(hi, W1O2_ref[...], preferred_element_type=jnp.float32))
    h1 = (FWpair - GW).reshape(RT, C)
    dst = jnp.stack([jnp.sum(h1, axis=0), jnp.sum(h1 * h1, axis=0)])

    S = lo * _rep8(FED_ref[0], C) + hi * _rep8(FOD_ref[0], C)
    dots8 = jnp.dot(S, HALF8_ref[...], preferred_element_type=jnp.float32, precision=lax.Precision.HIGHEST)
    d3 = dots8[:, :2].reshape(TP, K // 2, 2)
    mm = jnp.max(jnp.max(d3, axis=1), axis=1)[:, None, None]
    e3 = jnp.exp(d3 - mm)
    ss = jnp.sum(jnp.sum(e3, axis=1), axis=1)[:, None, None]
    pp = (e3 / ss).reshape(RT2, 2)
    pp8 = jnp.concatenate([pp, jnp.zeros((RT2, 6), jnp.float32)], axis=1)
    wvec = jnp.dot(pp8, HALFT8_ref[...], preferred_element_type=jnp.float32, precision=lax.Precision.HIGHEST)
    TLs = jnp.sum((lo * wvec).reshape(TP, K // 2, C), axis=1)
    TOs = jnp.sum((hi * wvec).reshape(TP, K // 2, C), axis=1)
    wn_e = jnp.dot(TLs, FOLD_ref[...], preferred_element_type=jnp.float32, precision=lax.Precision.HIGHEST)
    wn_o = jnp.dot(TOs, FOLD_ref[...], preferred_element_type=jnp.float32, precision=lax.Precision.HIGHEST)
    wn = jnp.concatenate([wn_e, wn_o], axis=1)
    WN_ref[0] = wn
    sh1 = jnp.dot(wn, sw1tp_ref[...], preferred_element_type=jnp.float32, precision=lax.Precision.HIGHEST) + sb1_ref[...]
    sst = jnp.stack([jnp.sum(sh1, axis=0), jnp.sum(sh1 * sh1, axis=0)])

    @pl.when(t == 0)
    def _():
        DST_ref[0] = dst
        SST_ref[0] = sst

    @pl.when(t != 0)
    def _():
        DST_ref[0] += dst
        SST_ref[0] += sst


def _p2_body(Gp_ref, F_ref, WN_ref, dsc_ref, dsh_ref, ssc_ref, ssh_ref,
             w1t_ref, b1_ref, W1E2_ref, W1O2_ref,
             w2t_ref, b2_ref, sw1tp_ref, sb1_ref,
             sw2t_ref, sb2_ref, fw1t_ref, fb1_ref, FH_ref, FST_ref):
    t = pl.program_id(1)
    lo, hi = _lohi(Gp_ref[0])
    F = F_ref[0]
    FW = jnp.dot(F, w1t_ref[...], preferred_element_type=jnp.float32, precision=lax.Precision.HIGHEST) + b1_ref[...]
    FWpair = _rep8(jnp.concatenate([FW, FW], axis=1), 2 * C)
    GW = (jnp.dot(lo, W1E2_ref[...], preferred_element_type=jnp.float32)
          + jnp.dot(hi, W1O2_ref[...], preferred_element_type=jnp.float32))
    h1 = (FWpair - GW).reshape(RT, C)
    a = jnp.maximum(h1 * dsc_ref[0] + dsh_ref[0], 0.0)
    u = jnp.dot(a, w2t_ref[...], preferred_element_type=jnp.float32) + b2_ref[...]
    dmax = jnp.max(u.reshape(TP, K, C), axis=1)

    wn = WN_ref[0]
    sh1 = jnp.dot(wn, sw1tp_ref[...], preferred_element_type=jnp.float32, precision=lax.Precision.HIGHEST) + sb1_ref[...]
    sa = jnp.maximum(sh1 * ssc_ref[0] + ssh_ref[0], 0.0)
    sim = jnp.dot(sa, sw2t_ref[...], preferred_element_type=jnp.float32, precision=lax.Precision.HIGHEST) + sb2_ref[...]

    cc = jnp.concatenate([dmax, sim], axis=1)
    fh = jnp.dot(cc, fw1t_ref[...], preferred_element_type=jnp.float32) + fb1_ref[...]
    FH_ref[0] = fh
    fst = jnp.stack([jnp.sum(fh, axis=0), jnp.sum(fh * fh, axis=0)])

    @pl.when(t == 0)
    def _():
        FST_ref[0] = fst

    @pl.when(t != 0)
    def _():
        FST_ref[0] += fst


def _p3_body(FH_ref, fsc_ref, fsh_ref, fw2t_ref, fb2_ref, O_ref):
    fh = FH_ref[0]
    fa = jnp.maximum(fh * fsc_ref[0] + fsh_ref[0], 0.0)
    O_ref[0] = jnp.dot(fa, fw2t_ref[...], preferred_element_type=jnp.float32) + fb2_ref[...]


def _affine(sums, count, g, be):
    mean = sums[:, 0, :] / count
    var = sums[:, 1, :] / count - mean * mean
    scale = g[None, :] / jnp.sqrt(var + EPS)
    shift = be[None, :] - mean * scale
    return scale[:, None, :], shift[:, None, :]


def kernel(features_0, features_1, knn_idx_0_to_1, knn_idx_1_to_0,
           d_w1, d_b1, d_g, d_be, d_w2, d_b2,
           s_w1, s_b1, s_g, s_be, s_w2, s_b2,
           f_w1, f_b1, f_g, f_be, f_w2, f_b2):
    C2 = 2 * C
    tabf = jnp.concatenate(
        [features_1.reshape(BN, C), features_0.reshape(BN, C)], axis=0)
    scale = 32700.0 / jnp.maximum(jnp.max(jnp.abs(tabf)), 1e-30)
    inv = 1.0 / scale
    tabq = jnp.round(tabf * scale).astype(jnp.int32)
    tab = (tabq[:, 0::2] & 0xFFFF) | (tabq[:, 1::2] << 16)
    boff = (jnp.arange(B, dtype=jnp.int32) * N)[None, :, None, None]
    doff = (jnp.arange(2, dtype=jnp.int32) * BN)[:, None, None, None]
    idxg = (jnp.stack([knn_idx_0_to_1, knn_idx_1_to_0]) + boff + doff).reshape(2 * R)
    idxg = idxg.reshape(-1, CH // 2, 2).transpose(0, 2, 1).reshape(2 * R)
    F = jnp.stack([features_0.reshape(BN, C), features_1.reshape(BN, C)])
    Fe, Fo = F[:, :, 0::2], F[:, :, 1::2]
    FED = jnp.concatenate([Fe, Fe], axis=2) * inv
    FOD = jnp.concatenate([Fo, Fo], axis=2) * inv

    w1t, w2t = d_w1.T, d_w2.T
    sw1t, sw2t = s_w1.T, s_w2.T
    fw1t, fw2t = f_w1.T, f_w2.T
    b1r, b2r = d_b1[None, :], d_b2[None, :]
    sb1r, sb2r = s_b1[None, :], s_b2[None, :]
    fb1r, fb2r = f_b1[None, :], f_b2[None, :]

    perm = jnp.concatenate([jnp.arange(0, C, 2), jnp.arange(1, C, 2)])
    sw1tp = sw1t[perm, :]
    Zc = jnp.zeros((C // 2, C), jnp.float32)
    w1tE, w1tO = w1t[0::2, :], w1t[1::2, :]
    W1E2 = jnp.concatenate([jnp.concatenate([w1tE, Zc], 1),
                            jnp.concatenate([Zc, w1tE], 1)], 0) * inv
    W1O2 = jnp.concatenate([jnp.concatenate([w1tO, Zc], 1),
                            jnp.concatenate([Zc, w1tO], 1)], 0) * inv
    lane = jnp.arange(C)
    HALF8 = jnp.stack([(lane < C // 2).astype(jnp.float32),
                       (lane >= C // 2).astype(jnp.float32)] +
                      [jnp.zeros((C,), jnp.float32)] * 6, axis=1)
    HALFT8 = jnp.concatenate(
        [jnp.stack([(lane < C // 2).astype(jnp.float32),
                    (lane >= C // 2).astype(jnp.float32)]),
         jnp.zeros((6, C), jnp.float32)], axis=0)
    FOLD = (lane[:, None] % (C // 2) ==
            jnp.arange(C // 2)[None, :]).astype(jnp.float32) * inv

    G = _sc_gather(tab, idxg).reshape(2, R // 2, C)

    grid = (2, NT)
    cp = pltpu.CompilerParams(dimension_semantics=("arbitrary", "arbitrary"))
    full = lambda s: pl.BlockSpec(s, lambda d, t: (0,) * len(s))
    WN, DST, SST = pl.pallas_call(
        _p1_body,
        grid=grid,
        in_specs=[
            pl.BlockSpec((1, RT2, C), lambda d, t: (d, t, 0)),
            pl.BlockSpec((1, TP, C), lambda d, t: (d, t, 0)),
            pl.BlockSpec((1, TP, C), lambda d, t: (d, t, 0)),
            pl.BlockSpec((1, TP, C), lambda d, t: (d, t, 0)),
            full((C, C)), full((1, C)), full((C, C)), full((1, C)),
            full((C, C2)), full((C, C2)), full((C, 8)), full((8, C)),
            full((C, C // 2)),
        ],
        out_specs=[
            pl.BlockSpec((1, TP, C), lambda d, t: (d, t, 0)),
            pl.BlockSpec((1, 2, C), lambda d, t: (d, 0, 0)),
            pl.BlockSpec((1, 2, C), lambda d, t: (d, 0, 0)),
        ],
        out_shape=[
            jax.ShapeDtypeStruct((2, BN, C), jnp.float32),
            jax.ShapeDtypeStruct((2, 2, C), jnp.float32),
            jax.ShapeDtypeStruct((2, 2, C), jnp.float32),
        ],
        compiler_params=cp,
    )(G, F, FED, FOD, w1t, b1r, sw1tp, sb1r, W1E2, W1O2, HALF8, HALFT8, FOLD)

    dsc, dsh = _affine(DST, float(R), d_g, d_be)
    ssc, ssh = _affine(SST, float(BN), s_g, s_be)

    FH, FST = pl.pallas_call(
        _p2_body,
        grid=grid,
        in_specs=[
            pl.BlockSpec((1, RT2, C), lambda d, t: (d, t, 0)),
            pl.BlockSpec((1, TP, C), lambda d, t: (d, t, 0)),
            pl.BlockSpec((1, TP, C), lambda d, t: (d, t, 0)),
            pl.BlockSpec((1, 1, C), lambda d, t: (d, 0, 0)),
            pl.BlockSpec((1, 1, C), lambda d, t: (d, 0, 0)),
            pl.BlockSpec((1, 1, C), lambda d, t: (d, 0, 0)),
            pl.BlockSpec((1, 1, C), lambda d, t: (d, 0, 0)),
            full((C, C)), full((1, C)), full((C, C2)), full((C, C2)),
            full((C, C)), full((1, C)), full((C, C)), full((1, C)),
            full((C, C)), full((1, C)),
            full((C2, C2)), full((1, C2)),
        ],
        out_specs=[
            pl.BlockSpec((1, TP, C2), lambda d, t: (d, t, 0)),
            pl.BlockSpec((1, 2, C2), lambda d, t: (d, 0, 0)),
        ],
        out_shape=[
            jax.ShapeDtypeStruct((2, BN, C2), jnp.float32),
            jax.ShapeDtypeStruct((2, 2, C2), jnp.float32),
        ],
        compiler_params=cp,
    )(G, F, WN, dsc, dsh, ssc, ssh, w1t, b1r, W1E2, W1O2, w2t, b2r,
      sw1tp, sb1r, sw2t, sb2r, fw1t, fb1r)

    fsc, fsh = _affine(FST, float(BN), f_g, f_be)

    TP3 = 2048
    NT3 = BN // TP3
    OUT = pl.pallas_call(
        _p3_body,
        grid=(2, NT3),
        in_specs=[
            pl.BlockSpec((1, TP3, C2), lambda d, t: (d, t, 0)),
            pl.BlockSpec((1, 1, C2), lambda d, t: (d, 0, 0)),
            pl.BlockSpec((1, 1, C2), lambda d, t: (d, 0, 0)),
            full((C2, C)), full((1, C)),
        ],
        out_specs=[pl.BlockSpec((1, TP3, C), lambda d, t: (d, t, 0))],
        out_shape=[jax.ShapeDtypeStruct((2, BN, C), jnp.float32)],
        compiler_params=cp,
    )(FH, fsc, fsh, fw2t, fb2r)[0]

    return (OUT[0].reshape(B, N, C), OUT[1].reshape(B, N, C))


# speedup vs baseline: 1.1759x; 1.1759x over previous
import functools

import jax
import jax.numpy as jnp
from jax import lax
from jax.experimental import pallas as pl
from jax.experimental.pallas import tpu as pltpu
from jax.experimental.pallas import tpu_sc as plsc

EPS = 1e-5

B, N, K, C = 4, 4096, 16, 128
BN = B * N
R = BN * K
TP = 512
RT = TP * K
NT = BN // TP
CH = 128


def _sc_gather_body(nw, tab_hbm, idx_hbm, out_hbm, idxv, rows, g0, g1):
    rw = (2 * R) // nw
    nc = plsc.get_sparse_core_info().num_cores
    wid = lax.axis_index("s") * nc + lax.axis_index("c")
    base = wid * rw
    pltpu.sync_copy(idx_hbm.at[pl.ds(base, rw)], idxv)
    nj = (rw // CH) // 2
    H = CH // 2

    def gath(i, buf, sem):
        pltpu.async_copy(
            tab_hbm.at[idxv.at[pl.ds(i * CH, CH)]], rows.at[buf], sem)

    def gwait(i, buf, sem):
        pltpu.make_async_copy(
            tab_hbm.at[idxv.at[pl.ds(i * CH, CH)]], rows.at[buf], sem).wait()

    def wback(i, buf):
        ob = (base + i * CH) // 2
        pltpu.sync_copy(rows.at[buf, pl.ds(0, H)],
                        out_hbm.at[pl.ds(ob, H), pl.ds(0, C // 2)])
        pltpu.sync_copy(rows.at[buf, pl.ds(H, H)],
                        out_hbm.at[pl.ds(ob, H), pl.ds(C // 2, C // 2)])

    gath(0, 0, g0)

    def body(j, carry):
        i0 = 2 * j
        gwait(i0, 0, g0)
        gath(i0 + 1, 1, g1)
        wback(i0, 0)

        @pl.when(j + 1 < nj)
        def _():
            gath(i0 + 2, 0, g0)

        gwait(i0 + 1, 1, g1)
        wback(i0 + 1, 1)
        return carry

    lax.fori_loop(0, nj, body, 0)


def _sc_gather(tab, idxg):
    info = plsc.get_sparse_core_info()
    nw = info.num_cores * info.num_subcores
    rw = (2 * R) // nw
    mesh = plsc.VectorSubcoreMesh(core_axis_name="c", subcore_axis_name="s")
    f = pl.kernel(
        functools.partial(_sc_gather_body, nw),
        out_type=jax.ShapeDtypeStruct((R, C), jnp.int32),
        mesh=mesh,
        scratch_types=[
            pltpu.VMEM((rw,), jnp.int32),
            pltpu.VMEM((2, CH, C // 2), jnp.int32),
            pltpu.SemaphoreType.DMA,
            pltpu.SemaphoreType.DMA,
        ],
        compiler_params=pltpu.CompilerParams(use_tc_tiling_on_sc=False),
    )
    return f(tab, idxg)


RT2 = RT // 2


def _lohi(Gp):
    lo = ((Gp << 16) >> 16).astype(jnp.float32)
    hi = (Gp >> 16).astype(jnp.float32)
    return lo, hi


def _rep8(x, w):
    return jnp.broadcast_to(x[:, None, :], (TP, K // 2, w)).reshape(RT2, w)


def _p1_body(Gp_ref, F_ref, FED_ref, FOD_ref, w1t_ref, b1_ref,
             sw1tp_ref, sb1_ref, W1E2_ref, W1O2_ref, IV_ref,
             WN_ref, DST_ref, SST_ref):
    t = pl.program_id(1)
    lo, hi = _lohi(Gp_ref[0])
    F = F_ref[0]

    FW = jnp.dot(F, w1t_ref[...], preferred_element_type=jnp.float32, precision=lax.Precision.HIGHEST) + b1_ref[...]
    FWpair = _rep8(jnp.concatenate([FW, FW], axis=1), 2 * C)
    GW = (jnp.dot(lo, W1E2_ref[...], preferred_element_type=jnp.float32)
          + jnp.dot(hi, W1O2_ref[...], preferred_element_type=jnp.float32))
    h1 = (FWpair - GW).reshape(RT, C)
    dst = jnp.stack([jnp.sum(h1, axis=0), jnp.sum(h1 * h1, axis=0)])

    S = lo * _rep8(FED_ref[0], C) + hi * _rep8(FOD_ref[0], C)
    left = lax.broadcasted_iota(jnp.int32, (RT2, C), 1) < (C // 2)
    z = jnp.zeros_like(S)
    dots2 = jnp.concatenate(
        [jnp.sum(jnp.where(left, S, z), axis=1, keepdims=True),
         jnp.sum(jnp.where(left, z, S), axis=1, keepdims=True)], axis=1)
    d3 = dots2.reshape(TP, K // 2, 2)
    mm = jnp.max(jnp.max(d3, axis=1), axis=1)[:, None, None]
    e3 = jnp.exp(d3 - mm)
    ss = jnp.sum(jnp.sum(e3, axis=1), axis=1)[:, None, None]
    pp = (e3 / ss).reshape(RT2, 2)
    iv = IV_ref[0, 0]
    wvec = jnp.where(left, pp[:, 0:1], pp[:, 1:2])
    TLs = jnp.sum((lo * wvec).reshape(TP, K // 2, C), axis=1)
    TOs = jnp.sum((hi * wvec).reshape(TP, K // 2, C), axis=1)
    wn = jnp.concatenate([TLs[:, :C // 2] + TLs[:, C // 2:],
                          TOs[:, :C // 2] + TOs[:, C // 2:]], axis=1) * iv
    WN_ref[0] = wn
    sh1 = jnp.dot(wn, sw1tp_ref[...], preferred_element_type=jnp.float32) + sb1_ref[...]
    sst = jnp.stack([jnp.sum(sh1, axis=0), jnp.sum(sh1 * sh1, axis=0)])

    @pl.when(t == 0)
    def _():
        DST_ref[0] = dst
        SST_ref[0] = sst

    @pl.when(t != 0)
    def _():
        DST_ref[0] += dst
        SST_ref[0] += sst


def _p2_body(Gp_ref, F_ref, WN_ref, dsc_ref, dsh_ref, ssc_ref, ssh_ref,
             w1t_ref, b1_ref, W1E2_ref, W1O2_ref,
             w2t_ref, b2_ref, sw1tp_ref, sb1_ref,
             sw2t_ref, sb2_ref, fw1t_ref, fb1_ref, FH_ref, FST_ref):
    t = pl.program_id(1)
    lo, hi = _lohi(Gp_ref[0])
    F = F_ref[0]
    FW = jnp.dot(F, w1t_ref[...], preferred_element_type=jnp.float32, precision=lax.Precision.HIGHEST) + b1_ref[...]
    FWpair = _rep8(jnp.concatenate([FW, FW], axis=1), 2 * C)
    GW = (jnp.dot(lo, W1E2_ref[...], preferred_element_type=jnp.float32)
          + jnp.dot(hi, W1O2_ref[...], preferred_element_type=jnp.float32))
    h1 = (FWpair - GW).reshape(RT, C)
    a = jnp.maximum(h1 * dsc_ref[0] + dsh_ref[0], 0.0)
    u = jnp.dot(a, w2t_ref[...], preferred_element_type=jnp.float32) + b2_ref[...]
    dmax = jnp.max(u.reshape(TP, K, C), axis=1)

    wn = WN_ref[0]
    sh1 = jnp.dot(wn, sw1tp_ref[...], preferred_element_type=jnp.float32) + sb1_ref[...]
    sa = jnp.maximum(sh1 * ssc_ref[0] + ssh_ref[0], 0.0)
    sim = jnp.dot(sa, sw2t_ref[...], preferred_element_type=jnp.float32) + sb2_ref[...]

    cc = jnp.concatenate([dmax, sim], axis=1)
    fh = jnp.dot(cc, fw1t_ref[...], preferred_element_type=jnp.float32) + fb1_ref[...]
    FH_ref[0] = fh
    fst = jnp.stack([jnp.sum(fh, axis=0), jnp.sum(fh * fh, axis=0)])

    @pl.when(t == 0)
    def _():
        FST_ref[0] = fst

    @pl.when(t != 0)
    def _():
        FST_ref[0] += fst


def _p3_body(FH_ref, fsc_ref, fsh_ref, fw2t_ref, fb2_ref, O_ref):
    fh = FH_ref[0]
    fa = jnp.maximum(fh * fsc_ref[0] + fsh_ref[0], 0.0)
    O_ref[0] = jnp.dot(fa, fw2t_ref[...], preferred_element_type=jnp.float32) + fb2_ref[...]


def _affine(sums, count, g, be):
    mean = sums[:, 0, :] / count
    var = sums[:, 1, :] / count - mean * mean
    scale = g[None, :] / jnp.sqrt(var + EPS)
    shift = be[None, :] - mean * scale
    return scale[:, None, :], shift[:, None, :]


def kernel(features_0, features_1, knn_idx_0_to_1, knn_idx_1_to_0,
           d_w1, d_b1, d_g, d_be, d_w2, d_b2,
           s_w1, s_b1, s_g, s_be, s_w2, s_b2,
           f_w1, f_b1, f_g, f_be, f_w2, f_b2):
    C2 = 2 * C
    tabf = jnp.concatenate(
        [features_1.reshape(BN, C), features_0.reshape(BN, C)], axis=0)
    scale = 32700.0 / jnp.maximum(jnp.max(jnp.abs(tabf)), 1e-30)
    inv = 1.0 / scale
    tabq = jnp.round(tabf * scale).astype(jnp.int32)
    tab = (tabq[:, 0::2] & 0xFFFF) | (tabq[:, 1::2] << 16)
    boff = (jnp.arange(B, dtype=jnp.int32) * N)[None, :, None, None]
    doff = (jnp.arange(2, dtype=jnp.int32) * BN)[:, None, None, None]
    idxg = (jnp.stack([knn_idx_0_to_1, knn_idx_1_to_0]) + boff + doff).reshape(2 * R)
    idxg = idxg.reshape(-1, CH // 2, 2).transpose(0, 2, 1).reshape(2 * R)
    F = jnp.stack([features_0.reshape(BN, C), features_1.reshape(BN, C)])
    Fe, Fo = F[:, :, 0::2], F[:, :, 1::2]
    FED = jnp.concatenate([Fe, Fe], axis=2) * inv
    FOD = jnp.concatenate([Fo, Fo], axis=2) * inv

    w1t, w2t = d_w1.T, d_w2.T
    sw1t, sw2t = s_w1.T, s_w2.T
    fw1t, fw2t = f_w1.T, f_w2.T
    b1r, b2r = d_b1[None, :], d_b2[None, :]
    sb1r, sb2r = s_b1[None, :], s_b2[None, :]
    fb1r, fb2r = f_b1[None, :], f_b2[None, :]

    perm = jnp.concatenate([jnp.arange(0, C, 2), jnp.arange(1, C, 2)])
    sw1tp = sw1t[perm, :]
    Zc = jnp.zeros((C // 2, C), jnp.float32)
    w1tE, w1tO = w1t[0::2, :], w1t[1::2, :]
    W1E2 = jnp.concatenate([jnp.concatenate([w1tE, Zc], 1),
                            jnp.concatenate([Zc, w1tE], 1)], 0) * inv
    W1O2 = jnp.concatenate([jnp.concatenate([w1tO, Zc], 1),
                            jnp.concatenate([Zc, w1tO], 1)], 0) * inv
    lane = jnp.arange(C)
    IV = jnp.full((1, 1), inv, jnp.float32)

    G = _sc_gather(tab, idxg).reshape(2, R // 2, C)

    grid = (2, NT)
    cp = pltpu.CompilerParams(dimension_semantics=("arbitrary", "arbitrary"))
    full = lambda s: pl.BlockSpec(s, lambda d, t: (0,) * len(s))
    WN, DST, SST = pl.pallas_call(
        _p1_body,
        grid=grid,
        in_specs=[
            pl.BlockSpec((1, RT2, C), lambda d, t: (d, t, 0)),
            pl.BlockSpec((1, TP, C), lambda d, t: (d, t, 0)),
            pl.BlockSpec((1, TP, C), lambda d, t: (d, t, 0)),
            pl.BlockSpec((1, TP, C), lambda d, t: (d, t, 0)),
            full((C, C)), full((1, C)), full((C, C)), full((1, C)),
            full((C, C2)), full((C, C2)), full((1, 1)),
        ],
        out_specs=[
            pl.BlockSpec((1, TP, C), lambda d, t: (d, t, 0)),
            pl.BlockSpec((1, 2, C), lambda d, t: (d, 0, 0)),
            pl.BlockSpec((1, 2, C), lambda d, t: (d, 0, 0)),
        ],
        out_shape=[
            jax.ShapeDtypeStruct((2, BN, C), jnp.float32),
            jax.ShapeDtypeStruct((2, 2, C), jnp.float32),
            jax.ShapeDtypeStruct((2, 2, C), jnp.float32),
        ],
        compiler_params=cp,
    )(G, F, FED, FOD, w1t, b1r, sw1tp, sb1r, W1E2, W1O2, IV)

    dsc, dsh = _affine(DST, float(R), d_g, d_be)
    ssc, ssh = _affine(SST, float(BN), s_g, s_be)

    FH, FST = pl.pallas_call(
        _p2_body,
        grid=grid,
        in_specs=[
            pl.BlockSpec((1, RT2, C), lambda d, t: (d, t, 0)),
            pl.BlockSpec((1, TP, C), lambda d, t: (d, t, 0)),
            pl.BlockSpec((1, TP, C), lambda d, t: (d, t, 0)),
            pl.BlockSpec((1, 1, C), lambda d, t: (d, 0, 0)),
            pl.BlockSpec((1, 1, C), lambda d, t: (d, 0, 0)),
            pl.BlockSpec((1, 1, C), lambda d, t: (d, 0, 0)),
            pl.BlockSpec((1, 1, C), lambda d, t: (d, 0, 0)),
            full((C, C)), full((1, C)), full((C, C2)), full((C, C2)),
            full((C, C)), full((1, C)), full((C, C)), full((1, C)),
            full((C, C)), full((1, C)),
            full((C2, C2)), full((1, C2)),
        ],
        out_specs=[
            pl.BlockSpec((1, TP, C2), lambda d, t: (d, t, 0)),
            pl.BlockSpec((1, 2, C2), lambda d, t: (d, 0, 0)),
        ],
        out_shape=[
            jax.ShapeDtypeStruct((2, BN, C2), jnp.float32),
            jax.ShapeDtypeStruct((2, 2, C2), jnp.float32),
        ],
        compiler_params=cp,
    )(G, F, WN, dsc, dsh, ssc, ssh, w1t, b1r, W1E2, W1O2, w2t, b2r,
      sw1tp, sb1r, sw2t, sb2r, fw1t, fb1r)

    fsc, fsh = _affine(FST, float(BN), f_g, f_be)

    TP3 = 2048
    NT3 = BN // TP3
    OUT = pl.pallas_call(
        _p3_body,
        grid=(2, NT3),
        in_specs=[
            pl.BlockSpec((1, TP3, C2), lambda d, t: (d, t, 0)),
            pl.BlockSpec((1, 1, C2), lambda d, t: (d, 0, 0)),
            pl.BlockSpec((1, 1, C2), lambda d, t: (d, 0, 0)),
            full((C2, C)), full((1, C)),
        ],
        out_specs=[pl.BlockSpec((1, TP3, C), lambda d, t: (d, t, 0))],
        out_shape=[jax.ShapeDtypeStruct((2, BN, C), jnp.float32)],
        compiler_params=cp,
    )(FH, fsc, fsh, fw2t, fb2r)[0]

    return (OUT[0].reshape(B, N, C), OUT[1].reshape(B, N, C))


# speedup vs baseline: 1.1793x; 1.0029x over previous
import functools

import jax
import jax.numpy as jnp
from jax import lax
from jax.experimental import pallas as pl
from jax.experimental.pallas import tpu as pltpu
from jax.experimental.pallas import tpu_sc as plsc

EPS = 1e-5

B, N, K, C = 4, 4096, 16, 128
BN = B * N
R = BN * K
TP = 512
RT = TP * K
NT = BN // TP
CH = 128


def _sc_gather_body(nw, tab_hbm, idx_hbm, out_hbm, idxv, rows, g0, g1):
    rw = (2 * R) // nw
    nc = plsc.get_sparse_core_info().num_cores
    wid = lax.axis_index("s") * nc + lax.axis_index("c")
    base = wid * rw
    pltpu.sync_copy(idx_hbm.at[pl.ds(base, rw)], idxv)
    nj = (rw // CH) // 2
    H = CH // 2

    def gath(i, buf, sem):
        pltpu.async_copy(
            tab_hbm.at[idxv.at[pl.ds(i * CH, CH)]], rows.at[buf], sem)

    def gwait(i, buf, sem):
        pltpu.make_async_copy(
            tab_hbm.at[idxv.at[pl.ds(i * CH, CH)]], rows.at[buf], sem).wait()

    def wback(i, buf):
        ob = (base + i * CH) // 2
        pltpu.sync_copy(rows.at[buf, pl.ds(0, H)],
                        out_hbm.at[pl.ds(ob, H), pl.ds(0, C // 2)])
        pltpu.sync_copy(rows.at[buf, pl.ds(H, H)],
                        out_hbm.at[pl.ds(ob, H), pl.ds(C // 2, C // 2)])

    gath(0, 0, g0)

    def body(j, carry):
        i0 = 2 * j
        gwait(i0, 0, g0)
        gath(i0 + 1, 1, g1)
        wback(i0, 0)

        @pl.when(j + 1 < nj)
        def _():
            gath(i0 + 2, 0, g0)

        gwait(i0 + 1, 1, g1)
        wback(i0 + 1, 1)
        return carry

    lax.fori_loop(0, nj, body, 0)


def _sc_gather(tab, idxg):
    info = plsc.get_sparse_core_info()
    nw = info.num_cores * info.num_subcores
    rw = (2 * R) // nw
    mesh = plsc.VectorSubcoreMesh(core_axis_name="c", subcore_axis_name="s")
    f = pl.kernel(
        functools.partial(_sc_gather_body, nw),
        out_type=jax.ShapeDtypeStruct((R, C), jnp.int32),
        mesh=mesh,
        scratch_types=[
            pltpu.VMEM((rw,), jnp.int32),
            pltpu.VMEM((2, CH, C // 2), jnp.int32),
            pltpu.SemaphoreType.DMA,
            pltpu.SemaphoreType.DMA,
        ],
        compiler_params=pltpu.CompilerParams(use_tc_tiling_on_sc=False),
    )
    return f(tab, idxg)


RT2 = RT // 2


def _lohi(Gp):
    lo = ((Gp << 16) >> 16).astype(jnp.float32)
    hi = (Gp >> 16).astype(jnp.float32)
    return lo, hi


def _rep8(x, w):
    return jnp.broadcast_to(x[:, None, :], (TP, K // 2, w)).reshape(RT2, w)


def _p1_body(Gp_ref, F_ref, FED_ref, FOD_ref, w1t_ref, b1_ref,
             sw1tp_ref, sb1_ref, W1E2_ref, W1O2_ref, IV_ref,
             WN_ref, DST_ref, SST_ref):
    t = pl.program_id(1)
    lo, hi = _lohi(Gp_ref[0])
    F = F_ref[0]

    FW = jnp.dot(F, w1t_ref[...], preferred_element_type=jnp.float32) + b1_ref[...]
    FWpair = _rep8(jnp.concatenate([FW, FW], axis=1), 2 * C)
    GW = (jnp.dot(lo, W1E2_ref[...], preferred_element_type=jnp.float32)
          + jnp.dot(hi, W1O2_ref[...], preferred_element_type=jnp.float32))
    h1 = (FWpair - GW).reshape(RT, C)
    dst = jnp.stack([jnp.sum(h1, axis=0), jnp.sum(h1 * h1, axis=0)])

    S = lo * _rep8(FED_ref[0], C) + hi * _rep8(FOD_ref[0], C)
    left = lax.broadcasted_iota(jnp.int32, (RT2, C), 1) < (C // 2)
    z = jnp.zeros_like(S)
    dots2 = jnp.concatenate(
        [jnp.sum(jnp.where(left, S, z), axis=1, keepdims=True),
         jnp.sum(jnp.where(left, z, S), axis=1, keepdims=True)], axis=1)
    d3 = dots2.reshape(TP, K // 2, 2)
    mm = jnp.max(jnp.max(d3, axis=1), axis=1)[:, None, None]
    e3 = jnp.exp(d3 - mm)
    ss = jnp.sum(jnp.sum(e3, axis=1), axis=1)[:, None, None]
    pp = (e3 / ss).reshape(RT2, 2)
    iv = IV_ref[0, 0]
    wvec = jnp.where(left, pp[:, 0:1], pp[:, 1:2])
    TLs = jnp.sum((lo * wvec).reshape(TP, K // 2, C), axis=1)
    TOs = jnp.sum((hi * wvec).reshape(TP, K // 2, C), axis=1)
    wn = jnp.concatenate([TLs[:, :C // 2] + TLs[:, C // 2:],
                          TOs[:, :C // 2] + TOs[:, C // 2:]], axis=1) * iv
    WN_ref[0] = wn
    sh1 = jnp.dot(wn, sw1tp_ref[...], preferred_element_type=jnp.float32) + sb1_ref[...]
    sst = jnp.stack([jnp.sum(sh1, axis=0), jnp.sum(sh1 * sh1, axis=0)])

    @pl.when(t == 0)
    def _():
        DST_ref[0] = dst
        SST_ref[0] = sst

    @pl.when(t != 0)
    def _():
        DST_ref[0] += dst
        SST_ref[0] += sst


def _p2_body(Gp_ref, F_ref, WN_ref, dsc_ref, dsh_ref, ssc_ref, ssh_ref,
             w1t_ref, b1_ref, W1E2_ref, W1O2_ref,
             w2t_ref, b2_ref, sw1tp_ref, sb1_ref,
             sw2t_ref, sb2_ref, fw1t_ref, fb1_ref, FH_ref, FST_ref):
    t = pl.program_id(1)
    lo, hi = _lohi(Gp_ref[0])
    F = F_ref[0]
    FW = jnp.dot(F, w1t_ref[...], preferred_element_type=jnp.float32) + b1_ref[...]
    FWpair = _rep8(jnp.concatenate([FW, FW], axis=1), 2 * C)
    GW = (jnp.dot(lo, W1E2_ref[...], preferred_element_type=jnp.float32)
          + jnp.dot(hi, W1O2_ref[...], preferred_element_type=jnp.float32))
    h1 = (FWpair - GW).reshape(RT, C)
    a = jnp.maximum(h1 * dsc_ref[0] + dsh_ref[0], 0.0)
    u = jnp.dot(a, w2t_ref[...], preferred_element_type=jnp.float32) + b2_ref[...]
    dmax = jnp.max(u.reshape(TP, K, C), axis=1)

    wn = WN_ref[0]
    sh1 = jnp.dot(wn, sw1tp_ref[...], preferred_element_type=jnp.float32) + sb1_ref[...]
    sa = jnp.maximum(sh1 * ssc_ref[0] + ssh_ref[0], 0.0)
    sim = jnp.dot(sa, sw2t_ref[...], preferred_element_type=jnp.float32) + sb2_ref[...]

    cc = jnp.concatenate([dmax, sim], axis=1)
    fh = jnp.dot(cc, fw1t_ref[...], preferred_element_type=jnp.float32) + fb1_ref[...]
    FH_ref[0] = fh
    fst = jnp.stack([jnp.sum(fh, axis=0), jnp.sum(fh * fh, axis=0)])

    @pl.when(t == 0)
    def _():
        FST_ref[0] = fst

    @pl.when(t != 0)
    def _():
        FST_ref[0] += fst


def _p3_body(FH_ref, fsc_ref, fsh_ref, fw2t_ref, fb2_ref, O_ref):
    fh = FH_ref[0]
    fa = jnp.maximum(fh * fsc_ref[0] + fsh_ref[0], 0.0)
    O_ref[0] = jnp.dot(fa, fw2t_ref[...], preferred_element_type=jnp.float32) + fb2_ref[...]


def _affine(sums, count, g, be):
    mean = sums[:, 0, :] / count
    var = sums[:, 1, :] / count - mean * mean
    scale = g[None, :] / jnp.sqrt(var + EPS)
    shift = be[None, :] - mean * scale
    return scale[:, None, :], shift[:, None, :]


def kernel(features_0, features_1, knn_idx_0_to_1, knn_idx_1_to_0,
           d_w1, d_b1, d_g, d_be, d_w2, d_b2,
           s_w1, s_b1, s_g, s_be, s_w2, s_b2,
           f_w1, f_b1, f_g, f_be, f_w2, f_b2):
    C2 = 2 * C
    tabf = jnp.concatenate(
        [features_1.reshape(BN, C), features_0.reshape(BN, C)], axis=0)
    scale = 32700.0 / jnp.maximum(jnp.max(jnp.abs(tabf)), 1e-30)
    inv = 1.0 / scale
    tabq = jnp.round(tabf * scale).astype(jnp.int32)
    tab = (tabq[:, 0::2] & 0xFFFF) | (tabq[:, 1::2] << 16)
    boff = (jnp.arange(B, dtype=jnp.int32) * N)[None, :, None, None]
    doff = (jnp.arange(2, dtype=jnp.int32) * BN)[:, None, None, None]
    idxg = (jnp.stack([knn_idx_0_to_1, knn_idx_1_to_0]) + boff + doff).reshape(2 * R)
    idxg = idxg.reshape(-1, CH // 2, 2).transpose(0, 2, 1).reshape(2 * R)
    F = jnp.stack([features_0.reshape(BN, C), features_1.reshape(BN, C)])
    Fe, Fo = F[:, :, 0::2], F[:, :, 1::2]
    FED = jnp.concatenate([Fe, Fe], axis=2) * inv
    FOD = jnp.concatenate([Fo, Fo], axis=2) * inv

    w1t, w2t = d_w1.T, d_w2.T
    sw1t, sw2t = s_w1.T, s_w2.T
    fw1t, fw2t = f_w1.T, f_w2.T
    b1r, b2r = d_b1[None, :], d_b2[None, :]
    sb1r, sb2r = s_b1[None, :], s_b2[None, :]
    fb1r, fb2r = f_b1[None, :], f_b2[None, :]

    perm = jnp.concatenate([jnp.arange(0, C, 2), jnp.arange(1, C, 2)])
    sw1tp = sw1t[perm, :]
    Zc = jnp.zeros((C // 2, C), jnp.float32)
    w1tE, w1tO = w1t[0::2, :], w1t[1::2, :]
    W1E2 = jnp.concatenate([jnp.concatenate([w1tE, Zc], 1),
                            jnp.concatenate([Zc, w1tE], 1)], 0) * inv
    W1O2 = jnp.concatenate([jnp.concatenate([w1tO, Zc], 1),
                            jnp.concatenate([Zc, w1tO], 1)], 0) * inv
    lane = jnp.arange(C)
    IV = jnp.full((1, 1), inv, jnp.float32)

    G = _sc_gather(tab, idxg).reshape(2, R // 2, C)

    grid = (2, NT)
    cp = pltpu.CompilerParams(dimension_semantics=("arbitrary", "arbitrary"))
    full = lambda s: pl.BlockSpec(s, lambda d, t: (0,) * len(s))
    WN, DST, SST = pl.pallas_call(
        _p1_body,
        grid=grid,
        in_specs=[
            pl.BlockSpec((1, RT2, C), lambda d, t: (d, t, 0)),
            pl.BlockSpec((1, TP, C), lambda d, t: (d, t, 0)),
            pl.BlockSpec((1, TP, C), lambda d, t: (d, t, 0)),
            pl.BlockSpec((1, TP, C), lambda d, t: (d, t, 0)),
            full((C, C)), full((1, C)), full((C, C)), full((1, C)),
            full((C, C2)), full((C, C2)), full((1, 1)),
        ],
        out_specs=[
            pl.BlockSpec((1, TP, C), lambda d, t: (d, t, 0)),
            pl.BlockSpec((1, 2, C), lambda d, t: (d, 0, 0)),
            pl.BlockSpec((1, 2, C), lambda d, t: (d, 0, 0)),
        ],
        out_shape=[
            jax.ShapeDtypeStruct((2, BN, C), jnp.float32),
            jax.ShapeDtypeStruct((2, 2, C), jnp.float32),
            jax.ShapeDtypeStruct((2, 2, C), jnp.float32),
        ],
        compiler_params=cp,
    )(G, F, FED, FOD, w1t, b1r, sw1tp, sb1r, W1E2, W1O2, IV)

    dsc, dsh = _affine(DST, float(R), d_g, d_be)
    ssc, ssh = _affine(SST, float(BN), s_g, s_be)

    FH, FST = pl.pallas_call(
        _p2_body,
        grid=grid,
        in_specs=[
            pl.BlockSpec((1, RT2, C), lambda d, t: (d, t, 0)),
            pl.BlockSpec((1, TP, C), lambda d, t: (d, t, 0)),
            pl.BlockSpec((1, TP, C), lambda d, t: (d, t, 0)),
            pl.BlockSpec((1, 1, C), lambda d, t: (d, 0, 0)),
            pl.BlockSpec((1, 1, C), lambda d, t: (d, 0, 0)),
            pl.BlockSpec((1, 1, C), lambda d, t: (d, 0, 0)),
            pl.BlockSpec((1, 1, C), lambda d, t: (d, 0, 0)),
            full((C, C)), full((1, C)), full((C, C2)), full((C, C2)),
            full((C, C)), full((1, C)), full((C, C)), full((1, C)),
            full((C, C)), full((1, C)),
            full((C2, C2)), full((1, C2)),
        ],
        out_specs=[
            pl.BlockSpec((1, TP, C2), lambda d, t: (d, t, 0)),
            pl.BlockSpec((1, 2, C2), lambda d, t: (d, 0, 0)),
        ],
        out_shape=[
            jax.ShapeDtypeStruct((2, BN, C2), jnp.float32),
            jax.ShapeDtypeStruct((2, 2, C2), jnp.float32),
        ],
        compiler_params=cp,
    )(G, F, WN, dsc, dsh, ssc, ssh, w1t, b1r, W1E2, W1O2, w2t, b2r,
      sw1tp, sb1r, sw2t, sb2r, fw1t, fb1r)

    fsc, fsh = _affine(FST, float(BN), f_g, f_be)

    TP3 = 2048
    NT3 = BN // TP3
    OUT = pl.pallas_call(
        _p3_body,
        grid=(2, NT3),
        in_specs=[
            pl.BlockSpec((1, TP3, C2), lambda d, t: (d, t, 0)),
            pl.BlockSpec((1, 1, C2), lambda d, t: (d, 0, 0)),
            pl.BlockSpec((1, 1, C2), lambda d, t: (d, 0, 0)),
            full((C2, C)), full((1, C)),
        ],
        out_specs=[pl.BlockSpec((1, TP3, C), lambda d, t: (d, t, 0))],
        out_shape=[jax.ShapeDtypeStruct((2, BN, C), jnp.float32)],
        compiler_params=cp,
    )(FH, fsc, fsh, fw2t, fb2r)[0]

    return (OUT[0].reshape(B, N, C), OUT[1].reshape(B, N, C))


# speedup vs baseline: 3.5319x; 2.9950x over previous
import functools

import jax
import jax.numpy as jnp
from jax import lax
from jax.experimental import pallas as pl
from jax.experimental.pallas import tpu as pltpu
from jax.experimental.pallas import tpu_sc as plsc

EPS = 1e-5

B, N, K, C = 4, 4096, 16, 128
BN = B * N
R = BN * K
TP = 512
RT = TP * K
NT = BN // TP
CH = 128


def _sc_gather_body(nw, tab_hbm, idx_hbm, out_hbm, idxv, rows, g0, g1):
    rw = (2 * R) // nw
    nc = plsc.get_sparse_core_info().num_cores
    wid = lax.axis_index("s") * nc + lax.axis_index("c")
    base = wid * rw
    pltpu.sync_copy(idx_hbm.at[pl.ds(base, rw)], idxv)
    nj = (rw // CH) // 2

    def gath(i, buf, sem):
        pltpu.async_copy(
            tab_hbm.at[idxv.at[pl.ds(i * CH, CH)]], rows.at[buf], sem)

    def gwait(i, buf, sem):
        pltpu.make_async_copy(
            tab_hbm.at[idxv.at[pl.ds(i * CH, CH)]], rows.at[buf], sem).wait()

    gath(0, 0, g0)

    def body(j, carry):
        i0 = 2 * j
        gwait(i0, 0, g0)
        gath(i0 + 1, 1, g1)
        pltpu.sync_copy(rows.at[0], out_hbm.at[pl.ds(base + i0 * CH, CH)])

        @pl.when(j + 1 < nj)
        def _():
            gath(i0 + 2, 0, g0)

        gwait(i0 + 1, 1, g1)
        pltpu.sync_copy(rows.at[1], out_hbm.at[pl.ds(base + (i0 + 1) * CH, CH)])
        return carry

    lax.fori_loop(0, nj, body, 0)


def _sc_gather(tab, idxg):
    info = plsc.get_sparse_core_info()
    nw = info.num_cores * info.num_subcores
    rw = (2 * R) // nw
    mesh = plsc.VectorSubcoreMesh(core_axis_name="c", subcore_axis_name="s")
    f = pl.kernel(
        functools.partial(_sc_gather_body, nw),
        out_type=jax.ShapeDtypeStruct((2 * R, C), jnp.float32),
        mesh=mesh,
        scratch_types=[
            pltpu.VMEM((rw,), jnp.int32),
            pltpu.VMEM((2, CH, C), jnp.float32),
            pltpu.SemaphoreType.DMA,
            pltpu.SemaphoreType.DMA,
        ],
    )
    return f(tab, idxg)


def _p1_body(G_ref, F_ref, w1t_ref, b1_ref, sw1t_ref, sb1_ref,
             H1_ref, WN_ref, DST_ref, SST_ref):
    t = pl.program_id(1)
    G = G_ref[0]
    F = F_ref[0]
    G3 = G.reshape(TP, K, C)
    D = (F[:, None, :] - G3).reshape(RT, C)
    h1 = jnp.dot(D, w1t_ref[...], preferred_element_type=jnp.float32) + b1_ref[...]
    H1_ref[0] = h1.astype(jnp.bfloat16)
    dst = jnp.stack([jnp.sum(h1, axis=0), jnp.sum(h1 * h1, axis=0)])

    dots = jnp.sum(F[:, None, :] * G3, axis=-1)
    m = jnp.max(dots, axis=1, keepdims=True)
    e = jnp.exp(dots - m)
    p = e / jnp.sum(e, axis=1, keepdims=True)
    wn = jnp.sum(G3 * p[:, :, None], axis=1)
    WN_ref[0] = wn
    sh1 = jnp.dot(wn, sw1t_ref[...], preferred_element_type=jnp.float32) + sb1_ref[...]
    sst = jnp.stack([jnp.sum(sh1, axis=0), jnp.sum(sh1 * sh1, axis=0)])

    @pl.when(t == 0)
    def _():
        DST_ref[0] = dst
        SST_ref[0] = sst

    @pl.when(t != 0)
    def _():
        DST_ref[0] += dst
        SST_ref[0] += sst


def _p2_body(H1_ref, WN_ref, dsc_ref, dsh_ref, ssc_ref, ssh_ref,
             w2t_ref, b2_ref, sw1t_ref, sb1_ref, sw2t_ref, sb2_ref,
             fw1t_ref, fb1_ref, FH_ref, FST_ref):
    t = pl.program_id(1)
    h1 = H1_ref[0].astype(jnp.float32)
    a = jnp.maximum(h1 * dsc_ref[0] + dsh_ref[0], 0.0)
    u = jnp.dot(a, w2t_ref[...], preferred_element_type=jnp.float32) + b2_ref[...]
    dmax = jnp.max(u.reshape(TP, K, C), axis=1)

    wn = WN_ref[0]
    sh1 = jnp.dot(wn, sw1t_ref[...], preferred_element_type=jnp.float32) + sb1_ref[...]
    sa = jnp.maximum(sh1 * ssc_ref[0] + ssh_ref[0], 0.0)
    sim = jnp.dot(sa, sw2t_ref[...], preferred_element_type=jnp.float32) + sb2_ref[...]

    cc = jnp.concatenate([dmax, sim], axis=1)
    fh = jnp.dot(cc, fw1t_ref[...], preferred_element_type=jnp.float32) + fb1_ref[...]
    FH_ref[0] = fh
    fst = jnp.stack([jnp.sum(fh, axis=0), jnp.sum(fh * fh, axis=0)])

    @pl.when(t == 0)
    def _():
        FST_ref[0] = fst

    @pl.when(t != 0)
    def _():
        FST_ref[0] += fst


def _p3_body(FH_ref, fsc_ref, fsh_ref, fw2t_ref, fb2_ref, O_ref):
    fh = FH_ref[0]
    fa = jnp.maximum(fh * fsc_ref[0] + fsh_ref[0], 0.0)
    O_ref[0] = jnp.dot(fa, fw2t_ref[...], preferred_element_type=jnp.float32) + fb2_ref[...]


def _affine(sums, count, g, be):
    mean = sums[:, 0, :] / count
    var = sums[:, 1, :] / count - mean * mean
    scale = g[None, :] / jnp.sqrt(var + EPS)
    shift = be[None, :] - mean * scale
    return scale[:, None, :], shift[:, None, :]


def kernel(features_0, features_1, knn_idx_0_to_1, knn_idx_1_to_0,
           d_w1, d_b1, d_g, d_be, d_w2, d_b2,
           s_w1, s_b1, s_g, s_be, s_w2, s_b2,
           f_w1, f_b1, f_g, f_be, f_w2, f_b2):
    C2 = 2 * C
    tab = jnp.concatenate(
        [features_1.reshape(BN, C), features_0.reshape(BN, C)], axis=0)
    boff = (jnp.arange(B, dtype=jnp.int32) * N)[None, :, None, None]
    doff = (jnp.arange(2, dtype=jnp.int32) * BN)[:, None, None, None]
    idxg = (jnp.stack([knn_idx_0_to_1, knn_idx_1_to_0]) + boff + doff).reshape(2 * R)
    F = jnp.stack([features_0.reshape(BN, C), features_1.reshape(BN, C)])

    w1t, w2t = d_w1.T, d_w2.T
    sw1t, sw2t = s_w1.T, s_w2.T
    fw1t, fw2t = f_w1.T, f_w2.T
    b1r, b2r = d_b1[None, :], d_b2[None, :]
    sb1r, sb2r = s_b1[None, :], s_b2[None, :]
    fb1r, fb2r = f_b1[None, :], f_b2[None, :]

    G = _sc_gather(tab, idxg).reshape(2, R, C)

    grid = (2, NT)
    cp = pltpu.CompilerParams(dimension_semantics=("arbitrary", "arbitrary"))
    full = lambda s: pl.BlockSpec(s, lambda d, t: (0,) * len(s))
    H1, WN, DST, SST = pl.pallas_call(
        _p1_body,
        grid=grid,
        in_specs=[
            pl.BlockSpec((1, RT, C), lambda d, t: (d, t, 0)),
            pl.BlockSpec((1, TP, C), lambda d, t: (d, t, 0)),
            full((C, C)), full((1, C)), full((C, C)), full((1, C)),
        ],
        out_specs=[
            pl.BlockSpec((1, RT, C), lambda d, t: (d, t, 0)),
            pl.BlockSpec((1, TP, C), lambda d, t: (d, t, 0)),
            pl.BlockSpec((1, 2, C), lambda d, t: (d, 0, 0)),
            pl.BlockSpec((1, 2, C), lambda d, t: (d, 0, 0)),
        ],
        out_shape=[
            jax.ShapeDtypeStruct((2, R, C), jnp.bfloat16),
            jax.ShapeDtypeStruct((2, BN, C), jnp.float32),
            jax.ShapeDtypeStruct((2, 2, C), jnp.float32),
            jax.ShapeDtypeStruct((2, 2, C), jnp.float32),
        ],
        compiler_params=cp,
    )(G, F, w1t, b1r, sw1t, sb1r)

    dsc, dsh = _affine(DST, float(R), d_g, d_be)
    ssc, ssh = _affine(SST, float(BN), s_g, s_be)

    FH, FST = pl.pallas_call(
        _p2_body,
        grid=grid,
        in_specs=[
            pl.BlockSpec((1, RT, C), lambda d, t: (d, t, 0)),
            pl.BlockSpec((1, TP, C), lambda d, t: (d, t, 0)),
            pl.BlockSpec((1, 1, C), lambda d, t: (d, 0, 0)),
            pl.BlockSpec((1, 1, C), lambda d, t: (d, 0, 0)),
            pl.BlockSpec((1, 1, C), lambda d, t: (d, 0, 0)),
            pl.BlockSpec((1, 1, C), lambda d, t: (d, 0, 0)),
            full((C, C)), full((1, C)), full((C, C)), full((1, C)),
            full((C, C)), full((1, C)), full((C2, C2)), full((1, C2)),
        ],
        out_specs=[
            pl.BlockSpec((1, TP, C2), lambda d, t: (d, t, 0)),
            pl.BlockSpec((1, 2, C2), lambda d, t: (d, 0, 0)),
        ],
        out_shape=[
            jax.ShapeDtypeStruct((2, BN, C2), jnp.float32),
            jax.ShapeDtypeStruct((2, 2, C2), jnp.float32),
        ],
        compiler_params=cp,
    )(H1, WN, dsc, dsh, ssc, ssh, w2t, b2r, sw1t, sb1r, sw2t, sb2r, fw1t, fb1r)

    fsc, fsh = _affine(FST, float(BN), f_g, f_be)

    TP3 = 2048
    NT3 = BN // TP3
    OUT = pl.pallas_call(
        _p3_body,
        grid=(2, NT3),
        in_specs=[
            pl.BlockSpec((1, TP3, C2), lambda d, t: (d, t, 0)),
            pl.BlockSpec((1, 1, C2), lambda d, t: (d, 0, 0)),
            pl.BlockSpec((1, 1, C2), lambda d, t: (d, 0, 0)),
            full((C2, C)), full((1, C)),
        ],
        out_specs=[pl.BlockSpec((1, TP3, C), lambda d, t: (d, t, 0))],
        out_shape=[jax.ShapeDtypeStruct((2, BN, C), jnp.float32)],
        compiler_params=cp,
    )(FH, fsc, fsh, fw2t, fb2r)[0]

    return (OUT[0].reshape(B, N, C), OUT[1].reshape(B, N, C))


# speedup vs baseline: 3.9363x; 1.1145x over previous
import functools

import jax
import jax.numpy as jnp
from jax import lax
from jax.experimental import pallas as pl
from jax.experimental.pallas import tpu as pltpu
from jax.experimental.pallas import tpu_sc as plsc

EPS = 1e-5

B, N, K, C = 4, 4096, 16, 128
BN = B * N
R = BN * K
TP = 512
RT = TP * K
NT = BN // TP
CH = 128


def _sc_gather_body(nw, nrows, tab_hbm, idx_hbm, out_hbm, idxv, rows, g0, g1):
    rw = nrows // nw
    nc = plsc.get_sparse_core_info().num_cores
    wid = lax.axis_index("s") * nc + lax.axis_index("c")
    base = wid * rw
    pltpu.sync_copy(idx_hbm.at[pl.ds(base, rw)], idxv)
    nj = (rw // CH) // 2

    def gath(i, buf, sem):
        pltpu.async_copy(
            tab_hbm.at[idxv.at[pl.ds(i * CH, CH)]], rows.at[buf], sem)

    def gwait(i, buf, sem):
        pltpu.make_async_copy(
            tab_hbm.at[idxv.at[pl.ds(i * CH, CH)]], rows.at[buf], sem).wait()

    gath(0, 0, g0)

    def body(j, carry):
        i0 = 2 * j
        gwait(i0, 0, g0)
        gath(i0 + 1, 1, g1)
        pltpu.sync_copy(rows.at[0], out_hbm.at[pl.ds(base + i0 * CH, CH)])

        @pl.when(j + 1 < nj)
        def _():
            gath(i0 + 2, 0, g0)

        gwait(i0 + 1, 1, g1)
        pltpu.sync_copy(rows.at[1], out_hbm.at[pl.ds(base + (i0 + 1) * CH, CH)])
        return carry

    lax.fori_loop(0, nj, body, 0)


def _sc_gather(tab, idxg, nrows):
    info = plsc.get_sparse_core_info()
    nw = info.num_cores * info.num_subcores
    rw = nrows // nw
    mesh = plsc.VectorSubcoreMesh(core_axis_name="c", subcore_axis_name="s")
    f = pl.kernel(
        functools.partial(_sc_gather_body, nw, nrows),
        out_type=jax.ShapeDtypeStruct((nrows, C), jnp.float32),
        mesh=mesh,
        scratch_types=[
            pltpu.VMEM((rw,), jnp.int32),
            pltpu.VMEM((2, CH, C), jnp.float32),
            pltpu.SemaphoreType.DMA,
            pltpu.SemaphoreType.DMA,
        ],
    )
    return f(tab, idxg)


def _p1_body(G_ref, F_ref, w1t_ref, b1_ref, sw1t_ref, sb1_ref,
             H1_ref, WN_ref, DST_ref, SST_ref):
    t = pl.program_id(1)
    G = G_ref[0]
    F = F_ref[0]
    G3 = G.reshape(TP, K, C)
    D = (F[:, None, :] - G3).reshape(RT, C)
    h1 = jnp.dot(D, w1t_ref[...], preferred_element_type=jnp.float32) + b1_ref[...]
    H1_ref[0] = h1.astype(jnp.bfloat16)
    dst = jnp.stack([jnp.sum(h1, axis=0), jnp.sum(h1 * h1, axis=0)])

    dots = jnp.sum(F[:, None, :] * G3, axis=-1)
    m = jnp.max(dots, axis=1, keepdims=True)
    e = jnp.exp(dots - m)
    p = e / jnp.sum(e, axis=1, keepdims=True)
    wn = jnp.sum(G3 * p[:, :, None], axis=1)
    WN_ref[0] = wn
    sh1 = jnp.dot(wn, sw1t_ref[...], preferred_element_type=jnp.float32) + sb1_ref[...]
    sst = jnp.stack([jnp.sum(sh1, axis=0), jnp.sum(sh1 * sh1, axis=0)])

    @pl.when(t == 0)
    def _():
        DST_ref[0] = dst
        SST_ref[0] = sst

    @pl.when(t != 0)
    def _():
        DST_ref[0] += dst
        SST_ref[0] += sst


def _p2_body(H1_ref, WN_ref, dsc_ref, dsh_ref, ssc_ref, ssh_ref,
             w2t_ref, b2_ref, sw1t_ref, sb1_ref, sw2t_ref, sb2_ref,
             fw1t_ref, fb1_ref, FH_ref, FST_ref):
    t = pl.program_id(1)
    h1 = H1_ref[0].astype(jnp.float32)
    a = jnp.maximum(h1 * dsc_ref[0] + dsh_ref[0], 0.0)
    u = jnp.dot(a, w2t_ref[...], preferred_element_type=jnp.float32) + b2_ref[...]
    dmax = jnp.max(u.reshape(TP, K, C), axis=1)

    wn = WN_ref[0]
    sh1 = jnp.dot(wn, sw1t_ref[...], preferred_element_type=jnp.float32) + sb1_ref[...]
    sa = jnp.maximum(sh1 * ssc_ref[0] + ssh_ref[0], 0.0)
    sim = jnp.dot(sa, sw2t_ref[...], preferred_element_type=jnp.float32) + sb2_ref[...]

    cc = jnp.concatenate([dmax, sim], axis=1)
    fh = jnp.dot(cc, fw1t_ref[...], preferred_element_type=jnp.float32) + fb1_ref[...]
    FH_ref[0] = fh
    fst = jnp.stack([jnp.sum(fh, axis=0), jnp.sum(fh * fh, axis=0)])

    @pl.when(t == 0)
    def _():
        FST_ref[0] = fst

    @pl.when(t != 0)
    def _():
        FST_ref[0] += fst


def _p3_body(FH_ref, fsc_ref, fsh_ref, fw2t_ref, fb2_ref, O_ref):
    fh = FH_ref[0]
    fa = jnp.maximum(fh * fsc_ref[0] + fsh_ref[0], 0.0)
    O_ref[0] = jnp.dot(fa, fw2t_ref[...], preferred_element_type=jnp.float32) + fb2_ref[...]


def _affine(sums, count, g, be):
    mean = sums[:, 0, :] / count
    var = sums[:, 1, :] / count - mean * mean
    scale = g[None, :] / jnp.sqrt(var + EPS)
    shift = be[None, :] - mean * scale
    return scale[:, None, :], shift[:, None, :]


def kernel(features_0, features_1, knn_idx_0_to_1, knn_idx_1_to_0,
           d_w1, d_b1, d_g, d_be, d_w2, d_b2,
           s_w1, s_b1, s_g, s_be, s_w2, s_b2,
           f_w1, f_b1, f_g, f_be, f_w2, f_b2):
    C2 = 2 * C
    tab = jnp.concatenate(
        [features_1.reshape(BN, C), features_0.reshape(BN, C)], axis=0)
    boff = (jnp.arange(B, dtype=jnp.int32) * N)[None, :, None, None]
    doff = (jnp.arange(2, dtype=jnp.int32) * BN)[:, None, None, None]
    idxg = (jnp.stack([knn_idx_0_to_1, knn_idx_1_to_0]) + boff + doff).reshape(2 * R)
    F = jnp.stack([features_0.reshape(BN, C), features_1.reshape(BN, C)])

    w1t, w2t = d_w1.T, d_w2.T
    sw1t, sw2t = s_w1.T, s_w2.T
    fw1t, fw2t = f_w1.T, f_w2.T
    b1r, b2r = d_b1[None, :], d_b2[None, :]
    sb1r, sb2r = s_b1[None, :], s_b2[None, :]
    fb1r, fb2r = f_b1[None, :], f_b2[None, :]

    G0 = _sc_gather(tab, idxg[:R], R).reshape(1, R, C)
    G1 = _sc_gather(tab, idxg[R:], R).reshape(1, R, C)

    outs = []
    for d in range(2):
        outs.append(_tc_pipeline(
            G0 if d == 0 else G1, F[d:d + 1],
            w1t, w2t, sw1t, sw2t, fw1t, fw2t,
            b1r, b2r, sb1r, sb2r, fb1r, fb2r,
            d_g, d_be, s_g, s_be, f_g, f_be))
    return (outs[0].reshape(B, N, C), outs[1].reshape(B, N, C))


def _tc_pipeline(G, Fd, w1t, w2t, sw1t, sw2t, fw1t, fw2t,
                 b1r, b2r, sb1r, sb2r, fb1r, fb2r,
                 d_g, d_be, s_g, s_be, f_g, f_be):
    C2 = 2 * C
    grid = (1, NT)
    cp = pltpu.CompilerParams(dimension_semantics=("arbitrary", "arbitrary"))
    full = lambda s: pl.BlockSpec(s, lambda d, t: (0,) * len(s))
    H1, WN, DST, SST = pl.pallas_call(
        _p1_body,
        grid=grid,
        in_specs=[
            pl.BlockSpec((1, RT, C), lambda d, t: (d, t, 0)),
            pl.BlockSpec((1, TP, C), lambda d, t: (d, t, 0)),
            full((C, C)), full((1, C)), full((C, C)), full((1, C)),
        ],
        out_specs=[
            pl.BlockSpec((1, RT, C), lambda d, t: (d, t, 0)),
            pl.BlockSpec((1, TP, C), lambda d, t: (d, t, 0)),
            pl.BlockSpec((1, 2, C), lambda d, t: (d, 0, 0)),
            pl.BlockSpec((1, 2, C), lambda d, t: (d, 0, 0)),
        ],
        out_shape=[
            jax.ShapeDtypeStruct((1, R, C), jnp.bfloat16),
            jax.ShapeDtypeStruct((1, BN, C), jnp.float32),
            jax.ShapeDtypeStruct((1, 2, C), jnp.float32),
            jax.ShapeDtypeStruct((1, 2, C), jnp.float32),
        ],
        compiler_params=cp,
    )(G, Fd, w1t, b1r, sw1t, sb1r)

    dsc, dsh = _affine(DST, float(R), d_g, d_be)
    ssc, ssh = _affine(SST, float(BN), s_g, s_be)

    FH, FST = pl.pallas_call(
        _p2_body,
        grid=grid,
        in_specs=[
            pl.BlockSpec((1, RT, C), lambda d, t: (d, t, 0)),
            pl.BlockSpec((1, TP, C), lambda d, t: (d, t, 0)),
            pl.BlockSpec((1, 1, C), lambda d, t: (d, 0, 0)),
            pl.BlockSpec((1, 1, C), lambda d, t: (d, 0, 0)),
            pl.BlockSpec((1, 1, C), lambda d, t: (d, 0, 0)),
            pl.BlockSpec((1, 1, C), lambda d, t: (d, 0, 0)),
            full((C, C)), full((1, C)), full((C, C)), full((1, C)),
            full((C, C)), full((1, C)), full((C2, C2)), full((1, C2)),
        ],
        out_specs=[
            pl.BlockSpec((1, TP, C2), lambda d, t: (d, t, 0)),
            pl.BlockSpec((1, 2, C2), lambda d, t: (d, 0, 0)),
        ],
        out_shape=[
            jax.ShapeDtypeStruct((1, BN, C2), jnp.float32),
            jax.ShapeDtypeStruct((1, 2, C2), jnp.float32),
        ],
        compiler_params=cp,
    )(H1, WN, dsc, dsh, ssc, ssh, w2t, b2r, sw1t, sb1r, sw2t, sb2r, fw1t, fb1r)

    fsc, fsh = _affine(FST, float(BN), f_g, f_be)

    TP3 = 2048
    NT3 = BN // TP3
    OUT = pl.pallas_call(
        _p3_body,
        grid=(1, NT3),
        in_specs=[
            pl.BlockSpec((1, TP3, C2), lambda d, t: (d, t, 0)),
            pl.BlockSpec((1, 1, C2), lambda d, t: (d, 0, 0)),
            pl.BlockSpec((1, 1, C2), lambda d, t: (d, 0, 0)),
            full((C2, C)), full((1, C)),
        ],
        out_specs=[pl.BlockSpec((1, TP3, C), lambda d, t: (d, t, 0))],
        out_shape=[jax.ShapeDtypeStruct((1, BN, C), jnp.float32)],
        compiler_params=cp,
    )(FH, fsc, fsh, fw2t, fb2r)[0]

    return OUT[0]


# speedup vs baseline: 4.1633x; 1.0577x over previous
import functools

import jax
import jax.numpy as jnp
from jax import lax
from jax.experimental import pallas as pl
from jax.experimental.pallas import tpu as pltpu
from jax.experimental.pallas import tpu_sc as plsc

EPS = 1e-5

B, N, K, C = 4, 4096, 16, 128
BN = B * N
R = BN * K
TP = 1024
RT = TP * K
NT = BN // TP
CH = 128


def _sc_gather_body(nw, nrows, tab_hbm, idx_hbm, out_hbm, idxv, rows, g0, g1):
    rw = nrows // nw
    nc = plsc.get_sparse_core_info().num_cores
    wid = lax.axis_index("s") * nc + lax.axis_index("c")
    base = wid * rw
    pltpu.sync_copy(idx_hbm.at[pl.ds(base, rw)], idxv)
    nj = (rw // CH) // 2

    def gath(i, buf, sem):
        pltpu.async_copy(
            tab_hbm.at[idxv.at[pl.ds(i * CH, CH)]], rows.at[buf], sem)

    def gwait(i, buf, sem):
        pltpu.make_async_copy(
            tab_hbm.at[idxv.at[pl.ds(i * CH, CH)]], rows.at[buf], sem).wait()

    gath(0, 0, g0)

    def body(j, carry):
        i0 = 2 * j
        gwait(i0, 0, g0)
        gath(i0 + 1, 1, g1)
        pltpu.sync_copy(rows.at[0], out_hbm.at[pl.ds(base + i0 * CH, CH)])

        @pl.when(j + 1 < nj)
        def _():
            gath(i0 + 2, 0, g0)

        gwait(i0 + 1, 1, g1)
        pltpu.sync_copy(rows.at[1], out_hbm.at[pl.ds(base + (i0 + 1) * CH, CH)])
        return carry

    lax.fori_loop(0, nj, body, 0)


def _sc_gather(tab, idxg, nrows):
    info = plsc.get_sparse_core_info()
    nw = info.num_cores * info.num_subcores
    rw = nrows // nw
    mesh = plsc.VectorSubcoreMesh(core_axis_name="c", subcore_axis_name="s")
    f = pl.kernel(
        functools.partial(_sc_gather_body, nw, nrows),
        out_type=jax.ShapeDtypeStruct((nrows, C), jnp.float32),
        mesh=mesh,
        scratch_types=[
            pltpu.VMEM((rw,), jnp.int32),
            pltpu.VMEM((2, CH, C), jnp.float32),
            pltpu.SemaphoreType.DMA,
            pltpu.SemaphoreType.DMA,
        ],
    )
    return f(tab, idxg)


def _p1_body(G_ref, F_ref, w1t_ref, b1_ref, sw1t_ref, sb1_ref,
             H1_ref, WN_ref, DST_ref, SST_ref):
    t = pl.program_id(1)
    G = G_ref[0]
    F = F_ref[0]
    G3 = G.reshape(TP, K, C)
    D = (F[:, None, :] - G3).reshape(RT, C)
    h1 = jnp.dot(D, w1t_ref[...], preferred_element_type=jnp.float32) + b1_ref[...]
    H1_ref[0] = h1.astype(jnp.bfloat16)
    dst = jnp.stack([jnp.sum(h1, axis=0), jnp.sum(h1 * h1, axis=0)])

    dots = jnp.sum(F[:, None, :] * G3, axis=-1)
    m = jnp.max(dots, axis=1, keepdims=True)
    e = jnp.exp(dots - m)
    p = e / jnp.sum(e, axis=1, keepdims=True)
    wn = jnp.sum(G3 * p[:, :, None], axis=1)
    WN_ref[0] = wn
    sh1 = jnp.dot(wn, sw1t_ref[...], preferred_element_type=jnp.float32) + sb1_ref[...]
    sst = jnp.stack([jnp.sum(sh1, axis=0), jnp.sum(sh1 * sh1, axis=0)])

    @pl.when(t == 0)
    def _():
        DST_ref[0] = dst
        SST_ref[0] = sst

    @pl.when(t != 0)
    def _():
        DST_ref[0] += dst
        SST_ref[0] += sst


def _p2_body(H1_ref, WN_ref, dsc_ref, dsh_ref, ssc_ref, ssh_ref,
             w2t_ref, b2_ref, sw1t_ref, sb1_ref, sw2t_ref, sb2_ref,
             fw1t_ref, fb1_ref, FH_ref, FST_ref):
    t = pl.program_id(1)
    h1 = H1_ref[0].astype(jnp.float32)
    a = jnp.maximum(h1 * dsc_ref[0] + dsh_ref[0], 0.0)
    u = jnp.dot(a, w2t_ref[...], preferred_element_type=jnp.float32) + b2_ref[...]
    dmax = jnp.max(u.reshape(TP, K, C), axis=1)

    wn = WN_ref[0]
    sh1 = jnp.dot(wn, sw1t_ref[...], preferred_element_type=jnp.float32) + sb1_ref[...]
    sa = jnp.maximum(sh1 * ssc_ref[0] + ssh_ref[0], 0.0)
    sim = jnp.dot(sa, sw2t_ref[...], preferred_element_type=jnp.float32) + sb2_ref[...]

    cc = jnp.concatenate([dmax, sim], axis=1)
    fh = jnp.dot(cc, fw1t_ref[...], preferred_element_type=jnp.float32) + fb1_ref[...]
    FH_ref[0] = fh
    fst = jnp.stack([jnp.sum(fh, axis=0), jnp.sum(fh * fh, axis=0)])

    @pl.when(t == 0)
    def _():
        FST_ref[0] = fst

    @pl.when(t != 0)
    def _():
        FST_ref[0] += fst


def _p3_body(FH_ref, fsc_ref, fsh_ref, fw2t_ref, fb2_ref, O_ref):
    fh = FH_ref[0]
    fa = jnp.maximum(fh * fsc_ref[0] + fsh_ref[0], 0.0)
    O_ref[0] = jnp.dot(fa, fw2t_ref[...], preferred_element_type=jnp.float32) + fb2_ref[...]


def _affine(sums, count, g, be):
    mean = sums[:, 0, :] / count
    var = sums[:, 1, :] / count - mean * mean
    scale = g[None, :] / jnp.sqrt(var + EPS)
    shift = be[None, :] - mean * scale
    return scale[:, None, :], shift[:, None, :]


def kernel(features_0, features_1, knn_idx_0_to_1, knn_idx_1_to_0,
           d_w1, d_b1, d_g, d_be, d_w2, d_b2,
           s_w1, s_b1, s_g, s_be, s_w2, s_b2,
           f_w1, f_b1, f_g, f_be, f_w2, f_b2):
    C2 = 2 * C
    tab = jnp.concatenate(
        [features_1.reshape(BN, C), features_0.reshape(BN, C)], axis=0)
    boff = (jnp.arange(B, dtype=jnp.int32) * N)[None, :, None, None]
    doff = (jnp.arange(2, dtype=jnp.int32) * BN)[:, None, None, None]
    idxg = (jnp.stack([knn_idx_0_to_1, knn_idx_1_to_0]) + boff + doff).reshape(2 * R)
    F = jnp.stack([features_0.reshape(BN, C), features_1.reshape(BN, C)])

    w1t, w2t = d_w1.T, d_w2.T
    sw1t, sw2t = s_w1.T, s_w2.T
    fw1t, fw2t = f_w1.T, f_w2.T
    b1r, b2r = d_b1[None, :], d_b2[None, :]
    sb1r, sb2r = s_b1[None, :], s_b2[None, :]
    fb1r, fb2r = f_b1[None, :], f_b2[None, :]

    G0 = _sc_gather(tab, idxg[:R], R).reshape(1, R, C)
    G1 = _sc_gather(tab, idxg[R:], R).reshape(1, R, C)

    outs = []
    for d in range(2):
        outs.append(_tc_pipeline(
            G0 if d == 0 else G1, F[d:d + 1],
            w1t, w2t, sw1t, sw2t, fw1t, fw2t,
            b1r, b2r, sb1r, sb2r, fb1r, fb2r,
            d_g, d_be, s_g, s_be, f_g, f_be))
    return (outs[0].reshape(B, N, C), outs[1].reshape(B, N, C))


def _tc_pipeline(G, Fd, w1t, w2t, sw1t, sw2t, fw1t, fw2t,
                 b1r, b2r, sb1r, sb2r, fb1r, fb2r,
                 d_g, d_be, s_g, s_be, f_g, f_be):
    C2 = 2 * C
    grid = (1, NT)
    cp = pltpu.CompilerParams(dimension_semantics=("arbitrary", "arbitrary"))
    full = lambda s: pl.BlockSpec(s, lambda d, t: (0,) * len(s))
    H1, WN, DST, SST = pl.pallas_call(
        _p1_body,
        grid=grid,
        in_specs=[
            pl.BlockSpec((1, RT, C), lambda d, t: (d, t, 0)),
            pl.BlockSpec((1, TP, C), lambda d, t: (d, t, 0)),
            full((C, C)), full((1, C)), full((C, C)), full((1, C)),
        ],
        out_specs=[
            pl.BlockSpec((1, RT, C), lambda d, t: (d, t, 0)),
            pl.BlockSpec((1, TP, C), lambda d, t: (d, t, 0)),
            pl.BlockSpec((1, 2, C), lambda d, t: (d, 0, 0)),
            pl.BlockSpec((1, 2, C), lambda d, t: (d, 0, 0)),
        ],
        out_shape=[
            jax.ShapeDtypeStruct((1, R, C), jnp.bfloat16),
            jax.ShapeDtypeStruct((1, BN, C), jnp.float32),
            jax.ShapeDtypeStruct((1, 2, C), jnp.float32),
            jax.ShapeDtypeStruct((1, 2, C), jnp.float32),
        ],
        compiler_params=cp,
    )(G, Fd, w1t, b1r, sw1t, sb1r)

    dsc, dsh = _affine(DST, float(R), d_g, d_be)
    ssc, ssh = _affine(SST, float(BN), s_g, s_be)

    FH, FST = pl.pallas_call(
        _p2_body,
        grid=grid,
        in_specs=[
            pl.BlockSpec((1, RT, C), lambda d, t: (d, t, 0)),
            pl.BlockSpec((1, TP, C), lambda d, t: (d, t, 0)),
            pl.BlockSpec((1, 1, C), lambda d, t: (d, 0, 0)),
            pl.BlockSpec((1, 1, C), lambda d, t: (d, 0, 0)),
            pl.BlockSpec((1, 1, C), lambda d, t: (d, 0, 0)),
            pl.BlockSpec((1, 1, C), lambda d, t: (d, 0, 0)),
            full((C, C)), full((1, C)), full((C, C)), full((1, C)),
            full((C, C)), full((1, C)), full((C2, C2)), full((1, C2)),
        ],
        out_specs=[
            pl.BlockSpec((1, TP, C2), lambda d, t: (d, t, 0)),
            pl.BlockSpec((1, 2, C2), lambda d, t: (d, 0, 0)),
        ],
        out_shape=[
            jax.ShapeDtypeStruct((1, BN, C2), jnp.float32),
            jax.ShapeDtypeStruct((1, 2, C2), jnp.float32),
        ],
        compiler_params=cp,
    )(H1, WN, dsc, dsh, ssc, ssh, w2t, b2r, sw1t, sb1r, sw2t, sb2r, fw1t, fb1r)

    fsc, fsh = _affine(FST, float(BN), f_g, f_be)

    TP3 = 2048
    NT3 = BN // TP3
    OUT = pl.pallas_call(
        _p3_body,
        grid=(1, NT3),
        in_specs=[
            pl.BlockSpec((1, TP3, C2), lambda d, t: (d, t, 0)),
            pl.BlockSpec((1, 1, C2), lambda d, t: (d, 0, 0)),
            pl.BlockSpec((1, 1, C2), lambda d, t: (d, 0, 0)),
            full((C2, C)), full((1, C)),
        ],
        out_specs=[pl.BlockSpec((1, TP3, C), lambda d, t: (d, t, 0))],
        out_shape=[jax.ShapeDtypeStruct((1, BN, C), jnp.float32)],
        compiler_params=cp,
    )(FH, fsc, fsh, fw2t, fb2r)[0]

    return OUT[0]
